# baseline (speedup 1.0000x reference)
import jax
import jax.numpy as jnp
from jax import lax
from jax.experimental import pallas as pl
from jax.experimental.pallas import tpu as pltpu
from jax.experimental.pallas import tpu_sc as plsc

B, L = 16384, 200
N = B * L
NC, NS = 2, 16
NW = NC * NS
S = 4096
CPR = B // S
NCHUNK = N // S
PER_W = NCHUNK // NW
GROUPS = S // 16
NBUF = 3


def _sc_body(dow_hbm, month_hbm, opex_hbm, qtr_hbm, dtab_hbm, mtab_hbm,
             out_hbm, dtab_v, mtab_v, dw0, dw1, dw2, mo0, mo1, mo2,
             oc0, oc1, oc2, si0, si1, si2, so0, so1, so2, sf):
    wid = lax.axis_index("s") * NC + lax.axis_index("c")
    dow_v = [dw0, dw1, dw2]
    month_v = [mo0, mo1, mo2]
    oc_v = [oc0, oc1, oc2]
    si = [si0, si1, si2]
    so = [so0, so1, so2]
    pltpu.sync_copy(dtab_hbm, dtab_v)
    pltpu.sync_copy(mtab_hbm, mtab_v)

    def rb(c):
        cid = wid * PER_W + c
        return cid // CPR, (cid % CPR) * S

    def issue_in(c):
        b = c % NBUF
        r, b0 = rb(c)
        return [
            pltpu.async_copy(dow_hbm.at[r, pl.ds(b0, S)], dow_v[b], si[b]),
            pltpu.async_copy(month_hbm.at[r, pl.ds(b0, S)], month_v[b], si[b]),
        ]

    in_h = {0: issue_in(0), 1: issue_in(1)}
    out_h = {}
    flag_h = []

    for c in range(PER_W):
        b = c % NBUF
        r, b0 = rb(c)
        for h in in_h.pop(c):
            h.wait()

        dw, mo, oc = dow_v[b], month_v[b], oc_v[b]

        @plsc.parallel_loop(0, GROUPS, unroll=8)
        def grp(g):
            p = g * 16
            d4 = dw[pl.ds(p, 16)] * 4
            m4 = mo[pl.ds(p, 16)] * 4
            for ch in range(4):
                oc[ch, pl.ds(p, 16)] = plsc.load_gather(dtab_v, [d4 + ch])
            for ch in range(4):
                oc[4 + ch, pl.ds(p, 16)] = plsc.load_gather(mtab_v, [m4 + ch])

        out_h[c] = [
            pltpu.async_copy(oc.at[ch], out_hbm.at[ch, r, pl.ds(b0, S)], so[b])
            for ch in range(8)
        ]
        flag_h.append(pltpu.async_copy(opex_hbm.at[r, pl.ds(b0, S)],
                                       out_hbm.at[8, r, pl.ds(b0, S)], sf))
        flag_h.append(pltpu.async_copy(qtr_hbm.at[r, pl.ds(b0, S)],
                                       out_hbm.at[9, r, pl.ds(b0, S)], sf))
        if c + 2 < PER_W:
            if c - 1 >= 0:
                for h in out_h.pop(c - 1):
                    h.wait()
            in_h[c + 2] = issue_in(c + 2)

    for c in sorted(out_h):
        for h in out_h[c]:
            h.wait()
    for h in flag_h:
        h.wait()


@jax.jit
def _run(dow_t, month_t, opex_t, qtr_t, dtab, mtab):
    mesh = plsc.VectorSubcoreMesh(core_axis_name="c", subcore_axis_name="s",
                                  num_cores=NC, num_subcores=NS)
    f = pl.kernel(
        _sc_body,
        out_type=jax.ShapeDtypeStruct((10, L, B), jnp.float32),
        mesh=mesh,
        scratch_types=[
            pltpu.VMEM((24,), jnp.float32),
            pltpu.VMEM((48,), jnp.float32),
            pltpu.VMEM((S,), jnp.int32),
            pltpu.VMEM((S,), jnp.int32),
            pltpu.VMEM((S,), jnp.int32),
            pltpu.VMEM((S,), jnp.int32),
            pltpu.VMEM((S,), jnp.int32),
            pltpu.VMEM((S,), jnp.int32),
            pltpu.VMEM((8, S), jnp.float32),
            pltpu.VMEM((8, S), jnp.float32),
            pltpu.VMEM((8, S), jnp.float32),
            pltpu.SemaphoreType.DMA,
            pltpu.SemaphoreType.DMA,
            pltpu.SemaphoreType.DMA,
            pltpu.SemaphoreType.DMA,
            pltpu.SemaphoreType.DMA,
            pltpu.SemaphoreType.DMA,
            pltpu.SemaphoreType.DMA,
        ],
        compiler_params=pltpu.CompilerParams(needs_layout_passes=False),
    )
    return f(dow_t, month_t, opex_t, qtr_t, dtab, mtab)


def kernel(dow, month, is_opex, is_qtr_end, dow_table, month_table):
    dow_t = dow.T.astype(jnp.int32)
    month_t = month.T.astype(jnp.int32)
    dtab = jnp.pad(dow_table.reshape(20), (0, 4))
    mtab = month_table.reshape(48)
    out = _run(dow_t, month_t, is_opex.T, is_qtr_end.T, dtab, mtab)
    return out.transpose(2, 1, 0)


# speedup vs baseline: 1.0133x; 1.0133x over previous
import jax
import jax.numpy as jnp
from jax import lax
from jax.experimental import pallas as pl
from jax.experimental.pallas import tpu as pltpu
from jax.experimental.pallas import tpu_sc as plsc

B, L = 16384, 200
N = B * L
NC, NS = 2, 16
NW = NC * NS
S = 4096
CPR = B // S
NCHUNK = N // S
PER_W = NCHUNK // NW
GROUPS = S // 16


def _sc_body(dow_hbm, month_hbm, opex_hbm, qtr_hbm, dtab_hbm, mtab_hbm,
             out_hbm, dtab_v, mtab_v, dow_v, month_v, oc_v):
    wid = lax.axis_index("s") * NC + lax.axis_index("c")
    pltpu.sync_copy(dtab_hbm, dtab_v)
    pltpu.sync_copy(mtab_hbm, mtab_v)

    def chunk(t, _):
        cid = wid * PER_W + t
        r = cid // CPR
        b0 = (cid % CPR) * S
        pltpu.sync_copy(dow_hbm.at[r, pl.ds(b0, S)], dow_v)
        pltpu.sync_copy(month_hbm.at[r, pl.ds(b0, S)], month_v)

        @plsc.parallel_loop(0, GROUPS, unroll=8)
        def grp(g):
            p = g * 16
            d4 = dow_v[pl.ds(p, 16)] * 4
            m4 = month_v[pl.ds(p, 16)] * 4
            for c in range(4):
                oc_v[c, pl.ds(p, 16)] = plsc.load_gather(dtab_v, [d4 + c])
            for c in range(4):
                oc_v[4 + c, pl.ds(p, 16)] = plsc.load_gather(mtab_v, [m4 + c])

        for c in range(8):
            pltpu.sync_copy(oc_v.at[c], out_hbm.at[c, r, pl.ds(b0, S)])
        pltpu.sync_copy(opex_hbm.at[r, pl.ds(b0, S)], out_hbm.at[8, r, pl.ds(b0, S)])
        pltpu.sync_copy(qtr_hbm.at[r, pl.ds(b0, S)], out_hbm.at[9, r, pl.ds(b0, S)])
        return 0

    lax.fori_loop(0, PER_W, chunk, 0)


@jax.jit
def _run(dow_t, month_t, opex_t, qtr_t, dtab, mtab):
    mesh = plsc.VectorSubcoreMesh(core_axis_name="c", subcore_axis_name="s",
                                  num_cores=NC, num_subcores=NS)
    f = pl.kernel(
        _sc_body,
        out_type=jax.ShapeDtypeStruct((10, L, B), jnp.float32),
        mesh=mesh,
        scratch_types=[
            pltpu.VMEM((24,), jnp.float32),
            pltpu.VMEM((48,), jnp.float32),
            pltpu.VMEM((S,), jnp.int32),
            pltpu.VMEM((S,), jnp.int32),
            pltpu.VMEM((8, S), jnp.float32),
        ],
        compiler_params=pltpu.CompilerParams(needs_layout_passes=False),
    )
    return f(dow_t, month_t, opex_t, qtr_t, dtab, mtab)


def kernel(dow, month, is_opex, is_qtr_end, dow_table, month_table):
    dow_t = dow.T.astype(jnp.int32)
    month_t = month.T.astype(jnp.int32)
    dtab = jnp.pad(dow_table.reshape(20), (0, 4))
    mtab = month_table.reshape(48)
    out = _run(dow_t, month_t, is_opex.T, is_qtr_end.T, dtab, mtab)
    return out.transpose(2, 1, 0)


# speedup vs baseline: 5.5234x; 5.4507x over previous
import jax
import jax.numpy as jnp
from jax import lax
from jax.experimental import pallas as pl
from jax.experimental.pallas import tpu as pltpu
from jax.experimental.pallas import tpu_sc as plsc

B, L = 16384, 200
N = B * L
NC, NS = 2, 16
NW = NC * NS
S = 4096
CPR = B // S
NCHUNK = N // S
PER_W = NCHUNK // NW
GROUPS = S // 16
ND, NF, NO = 3, 4, 2


def _sc_body(dow_hbm, month_hbm, opex_hbm, qtr_hbm, dtab_hbm, mtab_hbm,
             out_hbm, dtab_v, mtab_v,
             dw0, dw1, dw2, mo0, mo1, mo2,
             fl0, fl1, fl2, fl3, oc0, oc1,
             sd0, sd1, sd2, sf0, sf1, sf2, sf3, sg0, sg1, so0, so1):
    wid = lax.axis_index("s") * NC + lax.axis_index("c")
    dow_v = [dw0, dw1, dw2]
    month_v = [mo0, mo1, mo2]
    flag_v = [fl0, fl1, fl2, fl3]
    oc_v = [oc0, oc1]
    sd = [sd0, sd1, sd2]
    sf = [sf0, sf1, sf2, sf3]
    sg = [sg0, sg1]
    so = [so0, so1]
    pltpu.sync_copy(dtab_hbm, dtab_v)
    pltpu.sync_copy(mtab_hbm, mtab_v)

    def rb(c):
        cid = wid * PER_W + c
        return cid // CPR, (cid % CPR) * S

    def issue_in(c):
        bd, bf = c % ND, c % NF
        r, b0 = rb(c)
        return [
            pltpu.async_copy(dow_hbm.at[r, pl.ds(b0, S)], dow_v[bd], sd[bd]),
            pltpu.async_copy(month_hbm.at[r, pl.ds(b0, S)], month_v[bd], sd[bd]),
            pltpu.async_copy(opex_hbm.at[r, pl.ds(b0, S)], flag_v[bf].at[0], sf[bf]),
            pltpu.async_copy(qtr_hbm.at[r, pl.ds(b0, S)], flag_v[bf].at[1], sf[bf]),
        ]

    in_h = {0: issue_in(0), 1: issue_in(1)}
    oc_h, fl_h = {}, {}

    for c in range(PER_W):
        bd, bf, bo = c % ND, c % NF, c % NO
        r, b0 = rb(c)
        for h in in_h.pop(c):
            h.wait()
        fl_h[c] = [
            pltpu.async_copy(flag_v[bf].at[0], out_hbm.at[8, r, pl.ds(b0, S)],
                             sg[c % NO]),
            pltpu.async_copy(flag_v[bf].at[1], out_hbm.at[9, r, pl.ds(b0, S)],
                             sg[c % NO]),
        ]
        if c - 2 >= 0:
            for h in oc_h.pop(c - 2):
                h.wait()

        dw, mo, oc = dow_v[bd], month_v[bd], oc_v[bo]

        @plsc.parallel_loop(0, GROUPS, unroll=4)
        def grp(g):
            p = g * 16
            d4 = dw[pl.ds(p, 16)] * 4
            m4 = mo[pl.ds(p, 16)] * 4
            for ch in range(4):
                oc[ch, pl.ds(p, 16)] = plsc.load_gather(dtab_v, [d4 + ch])
            for ch in range(4):
                oc[4 + ch, pl.ds(p, 16)] = plsc.load_gather(mtab_v, [m4 + ch])

        oc_h[c] = [
            pltpu.async_copy(oc.at[ch], out_hbm.at[ch, r, pl.ds(b0, S)], so[bo])
            for ch in range(8)
        ]
        if c - 2 >= 0:
            for h in fl_h.pop(c - 2):
                h.wait()
        if c + 2 < PER_W:
            in_h[c + 2] = issue_in(c + 2)

    for hs in list(oc_h.values()) + list(fl_h.values()):
        for h in hs:
            h.wait()


@jax.jit
def _run(dow_t, month_t, opex_t, qtr_t, dtab, mtab):
    mesh = plsc.VectorSubcoreMesh(core_axis_name="c", subcore_axis_name="s",
                                  num_cores=NC, num_subcores=NS)
    f = pl.kernel(
        _sc_body,
        out_type=jax.ShapeDtypeStruct((10, L, B), jnp.float32),
        mesh=mesh,
        scratch_types=(
            [pltpu.VMEM((24,), jnp.float32), pltpu.VMEM((48,), jnp.float32)]
            + [pltpu.VMEM((S,), jnp.int32) for _ in range(6)]
            + [pltpu.VMEM((2, S), jnp.float32) for _ in range(4)]
            + [pltpu.VMEM((8, S), jnp.float32) for _ in range(2)]
            + [pltpu.SemaphoreType.DMA for _ in range(11)]
        ),
        compiler_params=pltpu.CompilerParams(needs_layout_passes=False),
    )
    return f(dow_t, month_t, opex_t, qtr_t, dtab, mtab)


def kernel(dow, month, is_opex, is_qtr_end, dow_table, month_table):
    dow_t = dow.T.astype(jnp.int32)
    month_t = month.T.astype(jnp.int32)
    dtab = jnp.pad(dow_table.reshape(20), (0, 4))
    mtab = month_table.reshape(48)
    out = _run(dow_t, month_t, is_opex.T, is_qtr_end.T, dtab, mtab)
    return out.transpose(2, 1, 0)


# speedup vs baseline: 5.5413x; 1.0032x over previous
import jax
import jax.numpy as jnp
from jax import lax
from jax.experimental import pallas as pl
from jax.experimental.pallas import tpu as pltpu
from jax.experimental.pallas import tpu_sc as plsc

B, L = 16384, 200
N = B * L
NC, NS = 2, 16
NW = NC * NS
S = 4096
CPR = B // S
NCHUNK = N // S
PER_W = NCHUNK // NW
GROUPS = S // 16
ND, NF, NO = 3, 4, 2


def _sc_body(dow_hbm, month_hbm, opex_hbm, qtr_hbm, dtab_hbm, mtab_hbm,
             out_hbm, dtab_v, mtab_v,
             dw0, dw1, dw2, mo0, mo1, mo2,
             fl0, fl1, fl2, fl3, oc0, oc1,
             sd0, sd1, sd2, sf0, sf1, sf2, sf3, sg0, sg1, so0, so1):
    wid = lax.axis_index("s") * NC + lax.axis_index("c")
    dow_v = [dw0, dw1, dw2]
    month_v = [mo0, mo1, mo2]
    flag_v = [fl0, fl1, fl2, fl3]
    oc_v = [oc0, oc1]
    sd = [sd0, sd1, sd2]
    sf = [sf0, sf1, sf2, sf3]
    sg = [sg0, sg1]
    so = [so0, so1]
    pltpu.sync_copy(dtab_hbm, dtab_v)
    pltpu.sync_copy(mtab_hbm, mtab_v)

    def rb(c):
        cid = wid * PER_W + c
        return cid // CPR, (cid % CPR) * S

    def issue_in(c):
        bd, bf = c % ND, c % NF
        r, b0 = rb(c)
        return [
            pltpu.async_copy(dow_hbm.at[r, pl.ds(b0, S)], dow_v[bd], sd[bd]),
            pltpu.async_copy(month_hbm.at[r, pl.ds(b0, S)], month_v[bd], sd[bd]),
            pltpu.async_copy(opex_hbm.at[r, pl.ds(b0, S)], flag_v[bf].at[0], sf[bf]),
            pltpu.async_copy(qtr_hbm.at[r, pl.ds(b0, S)], flag_v[bf].at[1], sf[bf]),
        ]

    in_h = {0: issue_in(0), 1: issue_in(1)}
    oc_h, fl_h = {}, {}

    for c in range(PER_W):
        bd, bf, bo = c % ND, c % NF, c % NO
        r, b0 = rb(c)
        for h in in_h.pop(c):
            h.wait()
        fl_h[c] = [
            pltpu.async_copy(flag_v[bf].at[0], out_hbm.at[8, r, pl.ds(b0, S)],
                             sg[c % NO]),
            pltpu.async_copy(flag_v[bf].at[1], out_hbm.at[9, r, pl.ds(b0, S)],
                             sg[c % NO]),
        ]
        if c - 2 >= 0:
            for h in oc_h.pop(c - 2):
                h.wait()
            for h in fl_h.pop(c - 2):
                h.wait()
        if c + 2 < PER_W:
            in_h[c + 2] = issue_in(c + 2)

        dw, mo, oc = dow_v[bd], month_v[bd], oc_v[bo]

        @plsc.parallel_loop(0, GROUPS, unroll=4)
        def grp(g):
            p = g * 16
            d4 = dw[pl.ds(p, 16)] * 4
            m4 = mo[pl.ds(p, 16)] * 4
            for ch in range(4):
                oc[ch, pl.ds(p, 16)] = plsc.load_gather(dtab_v, [d4 + ch])
            for ch in range(4):
                oc[4 + ch, pl.ds(p, 16)] = plsc.load_gather(mtab_v, [m4 + ch])

        oc_h[c] = [
            pltpu.async_copy(oc.at[ch], out_hbm.at[ch, r, pl.ds(b0, S)], so[bo])
            for ch in range(8)
        ]

    for hs in list(oc_h.values()) + list(fl_h.values()):
        for h in hs:
            h.wait()


@jax.jit
def _run(dow_t, month_t, opex_t, qtr_t, dtab, mtab):
    mesh = plsc.VectorSubcoreMesh(core_axis_name="c", subcore_axis_name="s",
                                  num_cores=NC, num_subcores=NS)
    f = pl.kernel(
        _sc_body,
        out_type=jax.ShapeDtypeStruct((10, L, B), jnp.float32),
        mesh=mesh,
        scratch_types=(
            [pltpu.VMEM((24,), jnp.float32), pltpu.VMEM((48,), jnp.float32)]
            + [pltpu.VMEM((S,), jnp.int32) for _ in range(6)]
            + [pltpu.VMEM((2, S), jnp.float32) for _ in range(4)]
            + [pltpu.VMEM((8, S), jnp.float32) for _ in range(2)]
            + [pltpu.SemaphoreType.DMA for _ in range(11)]
        ),
        compiler_params=pltpu.CompilerParams(needs_layout_passes=False),
    )
    return f(dow_t, month_t, opex_t, qtr_t, dtab, mtab)


def kernel(dow, month, is_opex, is_qtr_end, dow_table, month_table):
    dow_t = dow.T.astype(jnp.int32)
    month_t = month.T.astype(jnp.int32)
    dtab = jnp.pad(dow_table.reshape(20), (0, 4))
    mtab = month_table.reshape(48)
    out = _run(dow_t, month_t, is_opex.T, is_qtr_end.T, dtab, mtab)
    return out.transpose(2, 1, 0)


# speedup vs baseline: 6.4774x; 1.1689x over previous
import jax
import jax.numpy as jnp
from jax import lax
from jax.experimental import pallas as pl
from jax.experimental.pallas import tpu as pltpu
from jax.experimental.pallas import tpu_sc as plsc

B, L = 16384, 200
N = B * L
NC, NS = 2, 16
NW = NC * NS
S = 2048
CPR = B // S
NCHUNK = N // S
PER_W = NCHUNK // NW
GROUPS = S // 16
NB = 5
LOOK = 3


def _sc_body(dow_hbm, month_hbm, opex_hbm, qtr_hbm, dtab_hbm, mtab_hbm,
             out_hbm, dtab_v, mtab_v,
             dw0, dw1, dw2, dw3, dw4, mo0, mo1, mo2, mo3, mo4,
             fl0, fl1, fl2, fl3, fl4, oc0, oc1, oc2, oc3, oc4,
             sd0, sd1, sd2, sd3, sd4, sf0, sf1, sf2, sf3, sf4,
             sg0, sg1, sg2, sg3, sg4, so0, so1, so2, so3, so4):
    wid = lax.axis_index("s") * NC + lax.axis_index("c")
    dow_v = [dw0, dw1, dw2, dw3, dw4]
    month_v = [mo0, mo1, mo2, mo3, mo4]
    flag_v = [fl0, fl1, fl2, fl3, fl4]
    oc_v = [oc0, oc1, oc2, oc3, oc4]
    sd = [sd0, sd1, sd2, sd3, sd4]
    sf = [sf0, sf1, sf2, sf3, sf4]
    sg = [sg0, sg1, sg2, sg3, sg4]
    so = [so0, so1, so2, so3, so4]
    pltpu.sync_copy(dtab_hbm, dtab_v)
    pltpu.sync_copy(mtab_hbm, mtab_v)

    def rb(c):
        cid = wid * PER_W + c
        return cid // CPR, (cid % CPR) * S

    def in_descs(c, j):
        r, b0 = rb(c)
        return [
            pltpu.make_async_copy(dow_hbm.at[r, pl.ds(b0, S)], dow_v[j], sd[j]),
            pltpu.make_async_copy(month_hbm.at[r, pl.ds(b0, S)], month_v[j], sd[j]),
            pltpu.make_async_copy(opex_hbm.at[r, pl.ds(b0, S)], flag_v[j].at[0], sf[j]),
            pltpu.make_async_copy(qtr_hbm.at[r, pl.ds(b0, S)], flag_v[j].at[1], sf[j]),
        ]

    def flout_descs(c, j):
        r, b0 = rb(c)
        return [
            pltpu.make_async_copy(flag_v[j].at[0], out_hbm.at[8, r, pl.ds(b0, S)], sg[j]),
            pltpu.make_async_copy(flag_v[j].at[1], out_hbm.at[9, r, pl.ds(b0, S)], sg[j]),
        ]

    def ocout_descs(c, j):
        r, b0 = rb(c)
        return [
            pltpu.make_async_copy(oc_v[j].at[ch], out_hbm.at[ch, r, pl.ds(b0, S)], so[j])
            for ch in range(8)
        ]

    for c0 in range(LOOK):
        for d in in_descs(c0, c0 % NB):
            d.start()

    def outer(i, _):
        base = i * NB
        for j in range(NB):
            c = base + j
            for d in in_descs(c, j):
                d.wait()
            for d in flout_descs(c, j):
                d.start()
            jf = (j + LOOK) % NB

            @pl.when(c - (NB - LOOK) >= 0)
            def _():
                for d in flout_descs(c - (NB - LOOK), jf):
                    d.wait()

            @pl.when(c - NB >= 0)
            def _():
                for d in ocout_descs(c - NB, j):
                    d.wait()

            @pl.when(c + LOOK < PER_W)
            def _():
                for d in in_descs(c + LOOK, jf):
                    d.start()

            dw, mo, oc = dow_v[j], month_v[j], oc_v[j]

            @plsc.parallel_loop(0, GROUPS, unroll=4)
            def grp(g):
                p = g * 16
                d4 = dw[pl.ds(p, 16)] * 4
                m4 = mo[pl.ds(p, 16)] * 4
                for ch in range(4):
                    oc[ch, pl.ds(p, 16)] = plsc.load_gather(dtab_v, [d4 + ch])
                for ch in range(4):
                    oc[4 + ch, pl.ds(p, 16)] = plsc.load_gather(mtab_v, [m4 + ch])

            for d in ocout_descs(c, j):
                d.start()
        return 0

    lax.fori_loop(0, PER_W // NB, outer, 0)

    for c in range(PER_W - (NB - LOOK), PER_W):
        for d in flout_descs(c, c % NB):
            d.wait()
    for c in range(PER_W - NB, PER_W):
        for d in ocout_descs(c, c % NB):
            d.wait()


@jax.jit
def _run(dow_t, month_t, opex_t, qtr_t, dtab, mtab):
    mesh = plsc.VectorSubcoreMesh(core_axis_name="c", subcore_axis_name="s",
                                  num_cores=NC, num_subcores=NS)
    f = pl.kernel(
        _sc_body,
        out_type=jax.ShapeDtypeStruct((10, L, B), jnp.float32),
        mesh=mesh,
        scratch_types=(
            [pltpu.VMEM((24,), jnp.float32), pltpu.VMEM((48,), jnp.float32)]
            + [pltpu.VMEM((S,), jnp.int32) for _ in range(10)]
            + [pltpu.VMEM((2, S), jnp.float32) for _ in range(5)]
            + [pltpu.VMEM((8, S), jnp.float32) for _ in range(5)]
            + [pltpu.SemaphoreType.DMA for _ in range(20)]
        ),
        compiler_params=pltpu.CompilerParams(needs_layout_passes=False),
    )
    return f(dow_t, month_t, opex_t, qtr_t, dtab, mtab)


def kernel(dow, month, is_opex, is_qtr_end, dow_table, month_table):
    dow_t = dow.T.astype(jnp.int32)
    month_t = month.T.astype(jnp.int32)
    dtab = jnp.pad(dow_table.reshape(20), (0, 4))
    mtab = month_table.reshape(48)
    out = _run(dow_t, month_t, is_opex.T, is_qtr_end.T, dtab, mtab)
    return out.transpose(2, 1, 0)


# speedup vs baseline: 6.6703x; 1.0298x over previous
import jax
import jax.numpy as jnp
from jax import lax
from jax.experimental import pallas as pl
from jax.experimental.pallas import tpu as pltpu
from jax.experimental.pallas import tpu_sc as plsc

B, L = 16384, 200
N = B * L
NC, NS = 2, 16
NW = NC * NS
S = 2048
CPR = B // S
NCHUNK = N // S
PER_W = NCHUNK // NW
GROUPS = S // 16
NB = 5
LOOK = 3


def _sc_body(dow_hbm, month_hbm, opex_hbm, qtr_hbm, dtab_hbm, mtab_hbm,
             out_hbm, dtab_v, mtab_v,
             dw0, dw1, dw2, dw3, dw4, mo0, mo1, mo2, mo3, mo4,
             fl0, fl1, fl2, fl3, fl4, oc0, oc1, oc2, oc3, oc4,
             sd0, sd1, sd2, sd3, sd4, sf0, sf1, sf2, sf3, sf4,
             sg0, sg1, sg2, sg3, sg4, so0, so1, so2, so3, so4):
    wid = lax.axis_index("s") * NC + lax.axis_index("c")
    dow_v = [dw0, dw1, dw2, dw3, dw4]
    month_v = [mo0, mo1, mo2, mo3, mo4]
    flag_v = [fl0, fl1, fl2, fl3, fl4]
    oc_v = [oc0, oc1, oc2, oc3, oc4]
    sd = [sd0, sd1, sd2, sd3, sd4]
    sf = [sf0, sf1, sf2, sf3, sf4]
    sg = [sg0, sg1, sg2, sg3, sg4]
    so = [so0, so1, so2, so3, so4]
    pltpu.sync_copy(dtab_hbm, dtab_v)
    pltpu.sync_copy(mtab_hbm, mtab_v)

    def rb(c):
        cid = wid * PER_W + c
        return cid // CPR, (cid % CPR) * S

    def in_descs(c, j):
        r, b0 = rb(c)
        return [
            pltpu.make_async_copy(dow_hbm.at[r, pl.ds(b0, S)], dow_v[j], sd[j]),
            pltpu.make_async_copy(month_hbm.at[r, pl.ds(b0, S)], month_v[j], sd[j]),
            pltpu.make_async_copy(opex_hbm.at[r, pl.ds(b0, S)], flag_v[j].at[0], sf[j]),
            pltpu.make_async_copy(qtr_hbm.at[r, pl.ds(b0, S)], flag_v[j].at[1], sf[j]),
        ]

    def flout_descs(c, j):
        r, b0 = rb(c)
        return [
            pltpu.make_async_copy(flag_v[j].at[0], out_hbm.at[8, r, pl.ds(b0, S)], sg[j]),
            pltpu.make_async_copy(flag_v[j].at[1], out_hbm.at[9, r, pl.ds(b0, S)], sg[j]),
        ]

    def ocout_descs(c, j):
        r, b0 = rb(c)
        return [
            pltpu.make_async_copy(oc_v[j].at[ch], out_hbm.at[ch, r, pl.ds(b0, S)], so[j])
            for ch in range(8)
        ]

    for c0 in range(LOOK):
        for d in in_descs(c0, c0 % NB):
            d.start()

    def outer(i, _):
        base = i * NB
        for j in range(NB):
            c = base + j
            for d in in_descs(c, j):
                d.wait()
            for d in flout_descs(c, j):
                d.start()
            jf = (j + LOOK) % NB

            @pl.when(c - (NB - LOOK) >= 0)
            def _():
                for d in flout_descs(c - (NB - LOOK), jf):
                    d.wait()

            @pl.when(c - NB >= 0)
            def _():
                for d in ocout_descs(c - NB, j):
                    d.wait()

            @pl.when(c + LOOK < PER_W)
            def _():
                for d in in_descs(c + LOOK, jf):
                    d.start()

            dw, mo, oc = dow_v[j], month_v[j], oc_v[j]

            @plsc.parallel_loop(0, GROUPS, unroll=8)
            def grp(g):
                p = g * 16
                d4 = dw[pl.ds(p, 16)] * 4
                m4 = mo[pl.ds(p, 16)] * 4
                for ch in range(4):
                    oc[ch, pl.ds(p, 16)] = plsc.load_gather(dtab_v, [d4 + ch])
                for ch in range(4):
                    oc[4 + ch, pl.ds(p, 16)] = plsc.load_gather(mtab_v, [m4 + ch])

            for d in ocout_descs(c, j):
                d.start()
        return 0

    lax.fori_loop(0, PER_W // NB, outer, 0)

    for c in range(PER_W - (NB - LOOK), PER_W):
        for d in flout_descs(c, c % NB):
            d.wait()
    for c in range(PER_W - NB, PER_W):
        for d in ocout_descs(c, c % NB):
            d.wait()


@jax.jit
def _run(dow_t, month_t, opex_t, qtr_t, dtab, mtab):
    mesh = plsc.VectorSubcoreMesh(core_axis_name="c", subcore_axis_name="s",
                                  num_cores=NC, num_subcores=NS)
    f = pl.kernel(
        _sc_body,
        out_type=jax.ShapeDtypeStruct((10, L, B), jnp.float32),
        mesh=mesh,
        scratch_types=(
            [pltpu.VMEM((24,), jnp.float32), pltpu.VMEM((48,), jnp.float32)]
            + [pltpu.VMEM((S,), jnp.int32) for _ in range(10)]
            + [pltpu.VMEM((2, S), jnp.float32) for _ in range(5)]
            + [pltpu.VMEM((8, S), jnp.float32) for _ in range(5)]
            + [pltpu.SemaphoreType.DMA for _ in range(20)]
        ),
        compiler_params=pltpu.CompilerParams(needs_layout_passes=False),
    )
    return f(dow_t, month_t, opex_t, qtr_t, dtab, mtab)


def kernel(dow, month, is_opex, is_qtr_end, dow_table, month_table):
    dow_t = dow.T.astype(jnp.int32)
    month_t = month.T.astype(jnp.int32)
    dtab = jnp.pad(dow_table.reshape(20), (0, 4))
    mtab = month_table.reshape(48)
    out = _run(dow_t, month_t, is_opex.T, is_qtr_end.T, dtab, mtab)
    return out.transpose(2, 1, 0)
